# baseline (speedup 1.0000x reference)
import functools

import jax
import jax.numpy as jnp
from jax import lax
from jax.experimental import pallas as pl
from jax.experimental.pallas import tpu as pltpu
from jax.experimental.pallas import tpu_sc as plsc

_H = 64
_CHUNK = 128


_CS = 40


def _sc_repack_body(V, D, num_cores, num_workers,
                    emb_hbm, out_hbm, slab0, slab1, comp0, comp1,
                    r0, r1, wsem):
    wid = lax.axis_index("s") * num_cores + lax.axis_index("c")
    n_chunks = (V // 8) // _CS
    n_iters = (n_chunks + num_workers - 1) // num_workers
    slabs = (slab0, slab1)
    comps = (comp0, comp1)
    rsems = (r0, r1)

    def fire_read(i, s):
        cid = i * num_workers + wid

        @pl.when(cid < n_chunks)
        def _():
            pltpu.async_copy(emb_hbm.at[pl.ds(cid * _CS * 8, _CS * 8)],
                             slabs[s], rsems[s])

    for s in range(2):
        fire_read(s, s)

    def outer(io, carry):
        for s in range(2):
            i = io * 2 + s
            cid = i * num_workers + wid

            @pl.when(cid < n_chunks)
            def _proc():
                pltpu.make_async_copy(
                    emb_hbm.at[pl.ds(cid * _CS * 8, _CS * 8)],
                    slabs[s], rsems[s]).wait()

                @pl.when(i >= 2)
                def _wprev():
                    pltpu.make_async_copy(comps[s],
                                          out_hbm.at[pl.ds(0, _CS)],
                                          wsem).wait()

                for j in range(_CS):
                    for r in range(8):
                        comps[s][j, pl.ds(r * D, D)] = slabs[s][j * 8 + r, :]
                pltpu.async_copy(comps[s], out_hbm.at[pl.ds(cid * _CS, _CS)],
                                 wsem)

            fire_read(i + 2, s)
        return carry

    lax.fori_loop(0, n_iters // 2, outer, 0)
    for s in range(2):
        pltpu.make_async_copy(comps[s], out_hbm.at[pl.ds(0, _CS)],
                              wsem).wait()


@functools.lru_cache(maxsize=None)
def _make_sc_repack(V, D):
    info = plsc.get_sparse_core_info()
    nw = info.num_cores * info.num_subcores
    assert (V // 8) % _CS == 0
    mesh = plsc.VectorSubcoreMesh(core_axis_name="c", subcore_axis_name="s")
    body = functools.partial(_sc_repack_body, V, D, info.num_cores, nw)
    return pl.kernel(
        body,
        out_type=jax.ShapeDtypeStruct((V // 8, 8 * D), jnp.float32),
        mesh=mesh,
        scratch_types=[
            pltpu.VMEM((_CS * 8, D), jnp.float32),
            pltpu.VMEM((_CS * 8, D), jnp.float32),
            pltpu.VMEM((_CS, 8 * D), jnp.float32),
            pltpu.VMEM((_CS, 8 * D), jnp.float32),
            pltpu.SemaphoreType.DMA,
            pltpu.SemaphoreType.DMA,
            pltpu.SemaphoreType.DMA,
        ],
        compiler_params=pltpu.CompilerParams(use_tc_tiling_on_sc=True,
                                             needs_layout_passes=False),
    )


def _sc_gather_body(B, L, D, num_cores,
                    x_hbm, emb_hbm, out_hbm, x_v, xt_v, rows_v, gsem, ssem):
    wid = lax.axis_index("s") * num_cores + lax.axis_index("c")
    n_per_w = _CHUNK * L
    pltpu.sync_copy(x_hbm.at[pl.ds(wid * _CHUNK, _CHUNK)], x_v)

    def step(l, carry):
        col = jnp.full((16,), l, jnp.int32)
        for kg in range(_CHUNK // 16):
            rows = kg * 16 + lax.iota(jnp.int32, 16)
            xt_v[l, pl.ds(kg * 16, 16)] = plsc.load_gather(x_v, [rows, col])
        pltpu.async_copy(emb_hbm.at[xt_v.at[l]],
                         rows_v.at[pl.ds(l * _CHUNK, _CHUNK)], gsem)
        return carry

    lax.fori_loop(0, L, step, 0)
    pltpu.make_async_copy(emb_hbm.at[pl.ds(0, n_per_w)], rows_v, gsem).wait()

    def store(l, carry):
        pltpu.async_copy(
            rows_v.at[pl.ds(l * _CHUNK, _CHUNK)],
            out_hbm.at[pl.ds(l * B + wid * _CHUNK, _CHUNK), pl.ds(0, D)],
            ssem)
        return carry

    lax.fori_loop(0, L, store, 0)
    pltpu.make_async_copy(rows_v,
                          out_hbm.at[pl.ds(0, n_per_w), pl.ds(0, D)],
                          ssem).wait()


@functools.lru_cache(maxsize=None)
def _make_sc_gather(V, D, B, L):
    info = plsc.get_sparse_core_info()
    nw = info.num_cores * info.num_subcores
    assert B % (nw * _CHUNK) == 0 or B == nw * _CHUNK
    mesh = plsc.VectorSubcoreMesh(core_axis_name="c", subcore_axis_name="s")
    body = functools.partial(_sc_gather_body, B, L, D, info.num_cores)
    return pl.kernel(
        body,
        out_type=jax.ShapeDtypeStruct((B * L, 128), jnp.float32),
        mesh=mesh,
        scratch_types=[
            pltpu.VMEM((_CHUNK, L), jnp.int32),
            pltpu.VMEM((L, _CHUNK), jnp.int32),
            pltpu.VMEM((L * _CHUNK, D), jnp.float32),
            pltpu.SemaphoreType.DMA,
            pltpu.SemaphoreType.DMA,
        ],
        compiler_params=pltpu.CompilerParams(use_tc_tiling_on_sc=False,
                                             needs_layout_passes=False),
    )


def _lstm_body(L, xs_ref, mask_ref, wg_ref, bg_ref,
               w1_ref, b1_ref, w2_ref, b2_ref, w3_ref, b3_ref,
               w4_ref, b4_ref, w5_ref, b5_ref, out_ref, h_ref, c_ref):
    l = pl.program_id(0)

    @pl.when(l == 0)
    def _init():
        h_ref[...] = jnp.zeros_like(h_ref)
        c_ref[...] = jnp.zeros_like(c_ref)

    xt = xs_ref[0] * mask_ref[...]
    h = h_ref[...]
    cat = jnp.concatenate([xt, h], axis=1)
    g = (jnp.dot(cat, wg_ref[...], preferred_element_type=jnp.float32)
         + bg_ref[...])
    i_g = jax.nn.sigmoid(g[:, 0 * _H:1 * _H])
    f_g = jax.nn.sigmoid(g[:, 1 * _H:2 * _H])
    g_g = jnp.tanh(g[:, 2 * _H:3 * _H])
    o_g = jax.nn.sigmoid(g[:, 3 * _H:4 * _H])
    c_new = f_g * c_ref[...] + i_g * g_g
    h_new = o_g * jnp.tanh(c_new)
    h_ref[...] = h_new
    c_ref[...] = c_new

    @pl.when(l == L - 1)
    def _head():
        o = jax.nn.relu(h_new)
        o = jax.nn.relu(
            jnp.dot(o, w1_ref[...], preferred_element_type=jnp.float32)
            + b1_ref[...])
        o = jax.nn.relu(
            jnp.dot(o, w2_ref[...], preferred_element_type=jnp.float32)
            + b2_ref[...])
        o = jax.nn.relu(
            jnp.dot(o, w3_ref[...], preferred_element_type=jnp.float32)
            + b3_ref[...])
        o = jax.nn.relu(
            jnp.dot(o, w4_ref[...], preferred_element_type=jnp.float32)
            + b4_ref[...])
        z = jnp.sum(o * w5_ref[...], axis=1, keepdims=True) + b5_ref[...]
        out_ref[...] = jax.nn.sigmoid(z)


@functools.lru_cache(maxsize=None)
def _make_lstm(L, B, D):
    full = lambda shape: pl.BlockSpec(shape, lambda l: (0,) * len(shape))
    return pl.pallas_call(
        functools.partial(_lstm_body, L),
        grid=(L,),
        in_specs=[
            pl.BlockSpec((1, B, 128), lambda l: (l, 0, 0)),
            full((1, 128)),
            full((128 + _H, 4 * _H)),
            full((1, 4 * _H)),
            full((_H, 64)),
            full((1, 64)),
            full((64, 128)),
            full((1, 128)),
            full((128, 64)),
            full((1, 64)),
            full((64, 32)),
            full((1, 32)),
            full((1, 32)),
            full((1, 1)),
        ],
        out_specs=pl.BlockSpec((B, 1), lambda l: (0, 0)),
        out_shape=jax.ShapeDtypeStruct((B, 1), jnp.float32),
        scratch_shapes=[
            pltpu.VMEM((B, _H), jnp.float32),
            pltpu.VMEM((B, _H), jnp.float32),
        ],
        compiler_params=pltpu.CompilerParams(
            dimension_semantics=("arbitrary",)),
    )


def kernel(x, emb, W_ih, W_hh, b_ih, b_hh,
           W1, b1, W2, b2, W3, b3, W4, b4, W5, b5):
    B, L = x.shape
    V, D = emb.shape
    N = B * L

    repack = _make_sc_repack(V, D)
    emb_lin = repack(emb).reshape(V, D)

    gather = _make_sc_gather(V, D, B, L)
    e2d = gather(x.astype(jnp.int32), emb_lin)
    xs = e2d.reshape(L, B, 128)

    mask = (jnp.arange(128) < D).astype(jnp.float32).reshape(1, 128)
    wg = jnp.zeros((128 + _H, 4 * _H), jnp.float32)
    wg = wg.at[:D].set(jnp.transpose(W_ih)).at[128:].set(jnp.transpose(W_hh))

    lstm = _make_lstm(L, B, D)
    return lstm(
        xs,
        mask, wg,
        (b_ih + b_hh).reshape(1, 4 * _H),
        jnp.transpose(W1), b1.reshape(1, -1),
        jnp.transpose(W2), b2.reshape(1, -1),
        jnp.transpose(W3), b3.reshape(1, -1),
        jnp.transpose(W4), b4.reshape(1, -1),
        W5.reshape(1, -1), b5.reshape(1, 1),
    )


# speedup vs baseline: 1.0458x; 1.0458x over previous
import functools

import jax
import jax.numpy as jnp
from jax import lax
from jax.experimental import pallas as pl
from jax.experimental.pallas import tpu as pltpu
from jax.experimental.pallas import tpu_sc as plsc

_H = 64
_CHUNK = 128


_CS = 40


def _sc_repack_body(V, D, num_cores, num_workers,
                    emb_hbm, out_hbm, slab0, slab1, comp0, comp1,
                    r0, r1, wsem):
    wid = lax.axis_index("s") * num_cores + lax.axis_index("c")
    n_chunks = (V // 8) // _CS
    n_iters = (n_chunks + num_workers - 1) // num_workers
    slabs = (slab0, slab1)
    comps = (comp0, comp1)
    rsems = (r0, r1)

    def fire_read(i, s):
        cid = i * num_workers + wid

        @pl.when(cid < n_chunks)
        def _():
            pltpu.async_copy(emb_hbm.at[pl.ds(cid * _CS * 8, _CS * 8)],
                             slabs[s], rsems[s])

    for s in range(2):
        fire_read(s, s)

    def outer(io, carry):
        for s in range(2):
            i = io * 2 + s
            cid = i * num_workers + wid

            @pl.when(cid < n_chunks)
            def _proc():
                pltpu.make_async_copy(
                    emb_hbm.at[pl.ds(cid * _CS * 8, _CS * 8)],
                    slabs[s], rsems[s]).wait()

                @pl.when(i >= 2)
                def _wprev():
                    pltpu.make_async_copy(comps[s],
                                          out_hbm.at[pl.ds(0, _CS)],
                                          wsem).wait()

                for j in range(_CS):
                    for r in range(8):
                        comps[s][j, pl.ds(r * D, D)] = slabs[s][j * 8 + r, :]
                pltpu.async_copy(comps[s], out_hbm.at[pl.ds(cid * _CS, _CS)],
                                 wsem)

            fire_read(i + 2, s)
        return carry

    lax.fori_loop(0, n_iters // 2, outer, 0)
    for s in range(2):
        pltpu.make_async_copy(comps[s], out_hbm.at[pl.ds(0, _CS)],
                              wsem).wait()


@functools.lru_cache(maxsize=None)
def _make_sc_repack(V, D):
    info = plsc.get_sparse_core_info()
    nw = info.num_cores * info.num_subcores
    assert (V // 8) % _CS == 0
    mesh = plsc.VectorSubcoreMesh(core_axis_name="c", subcore_axis_name="s")
    body = functools.partial(_sc_repack_body, V, D, info.num_cores, nw)
    return pl.kernel(
        body,
        out_type=jax.ShapeDtypeStruct((V // 8, 8 * D), jnp.float32),
        mesh=mesh,
        scratch_types=[
            pltpu.VMEM((_CS * 8, D), jnp.float32),
            pltpu.VMEM((_CS * 8, D), jnp.float32),
            pltpu.VMEM((_CS, 8 * D), jnp.float32),
            pltpu.VMEM((_CS, 8 * D), jnp.float32),
            pltpu.SemaphoreType.DMA,
            pltpu.SemaphoreType.DMA,
            pltpu.SemaphoreType.DMA,
        ],
        compiler_params=pltpu.CompilerParams(use_tc_tiling_on_sc=True,
                                             needs_layout_passes=False),
    )


def _sc_gather_body(B, L, D, num_cores,
                    x_hbm, emb_hbm, out_hbm, x_v, xt_v, rows_v, gsem, ssem):
    wid = lax.axis_index("s") * num_cores + lax.axis_index("c")
    n_per_w = _CHUNK * L
    pltpu.sync_copy(x_hbm.at[pl.ds(wid * _CHUNK, _CHUNK)], x_v)

    def step(l, carry):
        col = jnp.full((16,), l, jnp.int32)
        for kg in range(_CHUNK // 16):
            rows = kg * 16 + lax.iota(jnp.int32, 16)
            xt_v[l, pl.ds(kg * 16, 16)] = plsc.load_gather(x_v, [rows, col])
        pltpu.async_copy(emb_hbm.at[xt_v.at[l]],
                         rows_v.at[pl.ds(l * _CHUNK, _CHUNK)], gsem)
        return carry

    lax.fori_loop(0, L, step, 0)
    pltpu.make_async_copy(emb_hbm.at[pl.ds(0, n_per_w)], rows_v, gsem).wait()

    def store(l, carry):
        pltpu.async_copy(
            rows_v.at[pl.ds(l * _CHUNK, _CHUNK)],
            out_hbm.at[pl.ds(l * B + wid * _CHUNK, _CHUNK), pl.ds(0, D)],
            ssem)
        return carry

    lax.fori_loop(0, L, store, 0)
    pltpu.make_async_copy(rows_v,
                          out_hbm.at[pl.ds(0, n_per_w), pl.ds(0, D)],
                          ssem).wait()


@functools.lru_cache(maxsize=None)
def _make_sc_gather(V, D, B, L):
    info = plsc.get_sparse_core_info()
    nw = info.num_cores * info.num_subcores
    assert B % (nw * _CHUNK) == 0 or B == nw * _CHUNK
    mesh = plsc.VectorSubcoreMesh(core_axis_name="c", subcore_axis_name="s")
    body = functools.partial(_sc_gather_body, B, L, D, info.num_cores)
    return pl.kernel(
        body,
        out_type=jax.ShapeDtypeStruct((B * L, 128), jnp.float32),
        mesh=mesh,
        scratch_types=[
            pltpu.VMEM((_CHUNK, L), jnp.int32),
            pltpu.VMEM((L, _CHUNK), jnp.int32),
            pltpu.VMEM((L * _CHUNK, D), jnp.float32),
            pltpu.SemaphoreType.DMA,
            pltpu.SemaphoreType.DMA,
        ],
        compiler_params=pltpu.CompilerParams(use_tc_tiling_on_sc=False,
                                             needs_layout_passes=False),
    )


def _lstm_body(L, xs_ref, mask_ref, wg_ref, bg_ref,
               w1_ref, b1_ref, w2_ref, b2_ref, w3_ref, b3_ref,
               w4_ref, b4_ref, w5_ref, b5_ref, out_ref, h_ref, c_ref):
    l = pl.program_id(0)

    @pl.when(l == 0)
    def _init():
        h_ref[...] = jnp.zeros_like(h_ref)
        c_ref[...] = jnp.zeros_like(c_ref)

    xt = xs_ref[0] * mask_ref[...]
    h = h_ref[...]
    cat = jnp.concatenate([xt, h], axis=1)
    g = (jnp.dot(cat, wg_ref[...], preferred_element_type=jnp.float32)
         + bg_ref[...])
    i_g = jax.nn.sigmoid(g[:, 0 * _H:1 * _H])
    f_g = jax.nn.sigmoid(g[:, 1 * _H:2 * _H])
    g_g = jnp.tanh(g[:, 2 * _H:3 * _H])
    o_g = jax.nn.sigmoid(g[:, 3 * _H:4 * _H])
    c_new = f_g * c_ref[...] + i_g * g_g
    h_new = o_g * jnp.tanh(c_new)
    h_ref[...] = h_new
    c_ref[...] = c_new

    @pl.when(l == L - 1)
    def _head():
        o = jax.nn.relu(h_new)
        o = jax.nn.relu(
            jnp.dot(o, w1_ref[...], preferred_element_type=jnp.float32)
            + b1_ref[...])
        o = jax.nn.relu(
            jnp.dot(o, w2_ref[...], preferred_element_type=jnp.float32)
            + b2_ref[...])
        o = jax.nn.relu(
            jnp.dot(o, w3_ref[...], preferred_element_type=jnp.float32)
            + b3_ref[...])
        o = jax.nn.relu(
            jnp.dot(o, w4_ref[...], preferred_element_type=jnp.float32)
            + b4_ref[...])
        z = jnp.sum(o * w5_ref[...], axis=1, keepdims=True) + b5_ref[...]
        out_ref[...] = jax.nn.sigmoid(z)


@functools.lru_cache(maxsize=None)
def _make_lstm(L, B, D):
    full = lambda shape: pl.BlockSpec(shape, lambda l: (0,) * len(shape))
    return pl.pallas_call(
        functools.partial(_lstm_body, L),
        grid=(L,),
        in_specs=[
            pl.BlockSpec((1, B, 128), lambda l: (l, 0, 0)),
            full((1, 128)),
            full((128 + _H, 4 * _H)),
            full((1, 4 * _H)),
            full((_H, 64)),
            full((1, 64)),
            full((64, 128)),
            full((1, 128)),
            full((128, 64)),
            full((1, 64)),
            full((64, 32)),
            full((1, 32)),
            full((1, 32)),
            full((1, 1)),
        ],
        out_specs=pl.BlockSpec((B, 1), lambda l: (0, 0)),
        out_shape=jax.ShapeDtypeStruct((B, 1), jnp.float32),
        scratch_shapes=[
            pltpu.VMEM((B, _H), jnp.float32),
            pltpu.VMEM((B, _H), jnp.float32),
        ],
        compiler_params=pltpu.CompilerParams(
            dimension_semantics=("arbitrary",)),
    )


def kernel(x, emb, W_ih, W_hh, b_ih, b_hh,
           W1, b1, W2, b2, W3, b3, W4, b4, W5, b5):
    B, L = x.shape
    V, D = emb.shape
    N = B * L

    emb128 = lax.optimization_barrier(emb.reshape(V // 8, 8 * D))
    emb_lin = emb128.reshape(V, D)

    gather = _make_sc_gather(V, D, B, L)
    e2d = gather(x.astype(jnp.int32), emb_lin)
    xs = e2d.reshape(L, B, 128)

    mask = (jnp.arange(128) < D).astype(jnp.float32).reshape(1, 128)
    wg = jnp.zeros((128 + _H, 4 * _H), jnp.float32)
    wg = wg.at[:D].set(jnp.transpose(W_ih)).at[128:].set(jnp.transpose(W_hh))

    lstm = _make_lstm(L, B, D)
    return lstm(
        xs,
        mask, wg,
        (b_ih + b_hh).reshape(1, 4 * _H),
        jnp.transpose(W1), b1.reshape(1, -1),
        jnp.transpose(W2), b2.reshape(1, -1),
        jnp.transpose(W3), b3.reshape(1, -1),
        jnp.transpose(W4), b4.reshape(1, -1),
        W5.reshape(1, -1), b5.reshape(1, 1),
    )


# speedup vs baseline: 1.3821x; 1.3216x over previous
import functools

import jax
import jax.numpy as jnp
from jax import lax
from jax.experimental import pallas as pl
from jax.experimental.pallas import tpu as pltpu
from jax.experimental.pallas import tpu_sc as plsc

_H = 64
_CHUNK = 128


_CS = 40


def _sc_repack_body(V, D, num_cores, num_workers,
                    emb_hbm, out_hbm, slab0, slab1, comp0, comp1,
                    r0, r1, wsem):
    wid = lax.axis_index("s") * num_cores + lax.axis_index("c")
    n_chunks = (V // 8) // _CS
    n_iters = (n_chunks + num_workers - 1) // num_workers
    slabs = (slab0, slab1)
    comps = (comp0, comp1)
    rsems = (r0, r1)

    def fire_read(i, s):
        cid = i * num_workers + wid

        @pl.when(cid < n_chunks)
        def _():
            pltpu.async_copy(emb_hbm.at[pl.ds(cid * _CS, _CS)],
                             slabs[s], rsems[s])

    for s in range(2):
        fire_read(s, s)

    def outer(io, carry):
        for s in range(2):
            i = io * 2 + s
            cid = i * num_workers + wid

            @pl.when(cid < n_chunks)
            def _proc():
                pltpu.make_async_copy(emb_hbm.at[pl.ds(cid * _CS, _CS)],
                                      slabs[s], rsems[s]).wait()

                @pl.when(i >= 2)
                def _wprev():
                    pltpu.make_async_copy(comps[s],
                                          out_hbm.at[pl.ds(0, _CS)],
                                          wsem).wait()

                for j in range(_CS):
                    for r in range(8):
                        comps[s][j, pl.ds(r * D, D)] = slabs[s][j, r, :]
                pltpu.async_copy(comps[s], out_hbm.at[pl.ds(cid * _CS, _CS)],
                                 wsem)

            fire_read(i + 2, s)
        return carry

    lax.fori_loop(0, n_iters // 2, outer, 0)
    for s in range(2):
        pltpu.make_async_copy(comps[s], out_hbm.at[pl.ds(0, _CS)],
                              wsem).wait()


@functools.lru_cache(maxsize=None)
def _make_sc_repack(V, D):
    info = plsc.get_sparse_core_info()
    nw = info.num_cores * info.num_subcores
    assert (V // 8) % _CS == 0
    mesh = plsc.VectorSubcoreMesh(core_axis_name="c", subcore_axis_name="s")
    body = functools.partial(_sc_repack_body, V, D, info.num_cores, nw)
    return pl.kernel(
        body,
        out_type=jax.ShapeDtypeStruct((V // 8, 8 * D), jnp.float32),
        mesh=mesh,
        scratch_types=[
            pltpu.VMEM((_CS, 8, D), jnp.float32),
            pltpu.VMEM((_CS, 8, D), jnp.float32),
            pltpu.VMEM((_CS, 8 * D), jnp.float32),
            pltpu.VMEM((_CS, 8 * D), jnp.float32),
            pltpu.SemaphoreType.DMA,
            pltpu.SemaphoreType.DMA,
            pltpu.SemaphoreType.DMA,
        ],
        compiler_params=pltpu.CompilerParams(use_tc_tiling_on_sc=True,
                                             needs_layout_passes=False),
    )


def _sc_gather_body(B, L, D, num_cores,
                    x_hbm, emb_hbm, out_hbm, x_v, xt_v, rows_v, gsem, ssem):
    wid = lax.axis_index("s") * num_cores + lax.axis_index("c")
    n_per_w = _CHUNK * L
    pltpu.sync_copy(x_hbm.at[pl.ds(wid * _CHUNK, _CHUNK)], x_v)

    def step(l, carry):
        col = jnp.full((16,), l, jnp.int32)
        for kg in range(_CHUNK // 16):
            rows = kg * 16 + lax.iota(jnp.int32, 16)
            xt_v[l, pl.ds(kg * 16, 16)] = plsc.load_gather(x_v, [rows, col])
        pltpu.async_copy(emb_hbm.at[xt_v.at[l]],
                         rows_v.at[pl.ds(l * _CHUNK, _CHUNK)], gsem)
        return carry

    lax.fori_loop(0, L, step, 0)
    pltpu.make_async_copy(emb_hbm.at[pl.ds(0, n_per_w)], rows_v, gsem).wait()

    def store(l, carry):
        pltpu.async_copy(
            rows_v.at[pl.ds(l * _CHUNK, _CHUNK)],
            out_hbm.at[pl.ds(l * B + wid * _CHUNK, _CHUNK), pl.ds(0, D)],
            ssem)
        return carry

    lax.fori_loop(0, L, store, 0)
    pltpu.make_async_copy(rows_v,
                          out_hbm.at[pl.ds(0, n_per_w), pl.ds(0, D)],
                          ssem).wait()


@functools.lru_cache(maxsize=None)
def _make_sc_gather(V, D, B, L):
    info = plsc.get_sparse_core_info()
    nw = info.num_cores * info.num_subcores
    assert B % (nw * _CHUNK) == 0 or B == nw * _CHUNK
    mesh = plsc.VectorSubcoreMesh(core_axis_name="c", subcore_axis_name="s")
    body = functools.partial(_sc_gather_body, B, L, D, info.num_cores)
    return pl.kernel(
        body,
        out_type=jax.ShapeDtypeStruct((B * L, 128), jnp.float32),
        mesh=mesh,
        scratch_types=[
            pltpu.VMEM((_CHUNK, L), jnp.int32),
            pltpu.VMEM((L, _CHUNK), jnp.int32),
            pltpu.VMEM((L * _CHUNK, D), jnp.float32),
            pltpu.SemaphoreType.DMA,
            pltpu.SemaphoreType.DMA,
        ],
        compiler_params=pltpu.CompilerParams(use_tc_tiling_on_sc=False,
                                             needs_layout_passes=False),
    )


def _lstm_body(L, xs_ref, mask_ref, wg_ref, bg_ref,
               w1_ref, b1_ref, w2_ref, b2_ref, w3_ref, b3_ref,
               w4_ref, b4_ref, w5_ref, b5_ref, out_ref, h_ref, c_ref):
    l = pl.program_id(0)

    @pl.when(l == 0)
    def _init():
        h_ref[...] = jnp.zeros_like(h_ref)
        c_ref[...] = jnp.zeros_like(c_ref)

    xt_t = jnp.swapaxes(xs_ref[0] * mask_ref[...], 0, 1)
    cat = jnp.concatenate([xt_t, h_ref[...]], axis=0)
    g = (jnp.dot(wg_ref[...], cat, preferred_element_type=jnp.float32)
         + bg_ref[...])
    i_g = jax.nn.sigmoid(g[0 * _H:1 * _H])
    f_g = jax.nn.sigmoid(g[1 * _H:2 * _H])
    g_g = jnp.tanh(g[2 * _H:3 * _H])
    o_g = jax.nn.sigmoid(g[3 * _H:4 * _H])
    c_new = f_g * c_ref[...] + i_g * g_g
    h_new = o_g * jnp.tanh(c_new)
    h_ref[...] = h_new
    c_ref[...] = c_new

    @pl.when(l == L - 1)
    def _head():
        o = jax.nn.relu(h_new)
        o = jax.nn.relu(
            jnp.dot(w1_ref[...], o, preferred_element_type=jnp.float32)
            + b1_ref[...])
        o = jax.nn.relu(
            jnp.dot(w2_ref[...], o, preferred_element_type=jnp.float32)
            + b2_ref[...])
        o = jax.nn.relu(
            jnp.dot(w3_ref[...], o, preferred_element_type=jnp.float32)
            + b3_ref[...])
        o = jax.nn.relu(
            jnp.dot(w4_ref[...], o, preferred_element_type=jnp.float32)
            + b4_ref[...])
        z = (jnp.dot(w5_ref[...], o, preferred_element_type=jnp.float32)
             + b5_ref[...])
        out_ref[...] = jax.nn.sigmoid(z)


@functools.lru_cache(maxsize=None)
def _make_lstm(L, B, D):
    full = lambda shape: pl.BlockSpec(shape, lambda l: (0,) * len(shape))
    return pl.pallas_call(
        functools.partial(_lstm_body, L),
        grid=(L,),
        in_specs=[
            pl.BlockSpec((1, B, 128), lambda l: (l, 0, 0)),
            full((1, 128)),
            full((4 * _H, 128 + _H)),
            full((4 * _H, 1)),
            full((64, _H)),
            full((64, 1)),
            full((128, 64)),
            full((128, 1)),
            full((64, 128)),
            full((64, 1)),
            full((32, 64)),
            full((32, 1)),
            full((1, 32)),
            full((1, 1)),
        ],
        out_specs=pl.BlockSpec((1, B), lambda l: (0, 0)),
        out_shape=jax.ShapeDtypeStruct((1, B), jnp.float32),
        scratch_shapes=[
            pltpu.VMEM((_H, B), jnp.float32),
            pltpu.VMEM((_H, B), jnp.float32),
        ],
        compiler_params=pltpu.CompilerParams(
            dimension_semantics=("arbitrary",)),
    )


def kernel(x, emb, W_ih, W_hh, b_ih, b_hh,
           W1, b1, W2, b2, W3, b3, W4, b4, W5, b5):
    B, L = x.shape
    V, D = emb.shape
    N = B * L

    repack = _make_sc_repack(V, D)
    emb_lin = repack(emb.reshape(V // 8, 8, D)).reshape(V, D)

    gather = _make_sc_gather(V, D, B, L)
    e2d = gather(x.astype(jnp.int32), emb_lin)
    xs = e2d.reshape(L, B, 128)

    mask = (jnp.arange(128) < D).astype(jnp.float32).reshape(1, 128)
    wg = jnp.zeros((4 * _H, 128 + _H), jnp.float32)
    wg = wg.at[:, :D].set(W_ih).at[:, 128:].set(W_hh)

    lstm = _make_lstm(L, B, D)
    out_t = lstm(
        xs,
        mask, wg,
        (b_ih + b_hh).reshape(4 * _H, 1),
        W1, b1.reshape(-1, 1),
        W2, b2.reshape(-1, 1),
        W3, b3.reshape(-1, 1),
        W4, b4.reshape(-1, 1),
        W5, b5.reshape(1, 1),
    )
    return out_t.reshape(B, 1)
